# Initial kernel scaffold; baseline (speedup 1.0000x reference)
import jax
import jax.numpy as jnp
from jax.experimental import pallas as pl


def kernel(goal_encoding, embed_table):
    raise NotImplementedError("write your pallas kernel here")


# speedup vs baseline: 1.7959x; 1.7959x over previous
import functools

import jax
import jax.numpy as jnp
from jax import lax
from jax.experimental import pallas as pl
from jax.experimental.pallas import tpu as pltpu
from jax.experimental.pallas import tpu_sc as plsc


@functools.cache
def _make_gather(B, V, D):
    info = plsc.get_sparse_core_info()
    NC, NS = info.num_cores, info.num_subcores
    NW = NC * NS
    assert B % NW == 0
    b_per_w = B // NW
    C = 512
    assert b_per_w % C == 0
    n_chunks = b_per_w // C

    mesh = plsc.VectorSubcoreMesh(core_axis_name="c", subcore_axis_name="s")

    @functools.partial(
        pl.kernel,
        mesh=mesh,
        compiler_params=pltpu.CompilerParams(use_tc_tiling_on_sc=False),
        out_type=jax.ShapeDtypeStruct((B, D), jnp.float32),
        scratch_types=[
            pltpu.VMEM((C,), jnp.int32),
            pltpu.VMEM((C, D), jnp.float32),
            pltpu.SemaphoreType.DMA,
        ],
    )
    def gather_kernel(idx_hbm, table_hbm, out_hbm, idx_v, rows_v, sem):
        wid = lax.axis_index("s") * NC + lax.axis_index("c")
        base = wid * b_per_w

        def body(i, carry):
            off = base + i * C
            pltpu.sync_copy(idx_hbm.at[pl.ds(off, C)], idx_v)
            pltpu.async_copy(table_hbm.at[idx_v], rows_v, sem).wait()
            pltpu.sync_copy(rows_v, out_hbm.at[pl.ds(off, C)])
            return carry

        lax.fori_loop(0, n_chunks, body, 0)

    return gather_kernel


def kernel(goal_encoding, embed_table):
    batch, hist = goal_encoding.shape
    v, d = embed_table.shape
    idx = goal_encoding.reshape(-1).astype(jnp.int32)
    out = _make_gather(batch * hist, v, d)(idx, embed_table)
    return out.reshape(batch, hist, d)


# speedup vs baseline: 1.8608x; 1.0361x over previous
import functools

import jax
import jax.numpy as jnp
from jax import lax
from jax.experimental import pallas as pl
from jax.experimental.pallas import tpu as pltpu
from jax.experimental.pallas import tpu_sc as plsc


@functools.cache
def _make_gather(B, V, D):
    info = plsc.get_sparse_core_info()
    NC, NS = info.num_cores, info.num_subcores
    NW = NC * NS
    assert B % NW == 0
    b_per_w = B // NW
    C = 512
    NBUF = 2
    assert b_per_w % (C * NBUF) == 0
    n_chunks = b_per_w // C
    n_groups = n_chunks // NBUF

    mesh = plsc.VectorSubcoreMesh(core_axis_name="c", subcore_axis_name="s")

    @functools.partial(
        pl.kernel,
        mesh=mesh,
        compiler_params=pltpu.CompilerParams(use_tc_tiling_on_sc=False),
        out_type=jax.ShapeDtypeStruct((B, D), jnp.float32),
        scratch_types=[
            pltpu.VMEM((b_per_w,), jnp.int32),
            pltpu.VMEM((NBUF, C, D), jnp.float32),
            [pltpu.SemaphoreType.DMA] * NBUF,
            [pltpu.SemaphoreType.DMA] * NBUF,
        ],
    )
    def gather_kernel(idx_hbm, table_hbm, out_hbm, idx_v, rows_v, gsems, osems):
        wid = lax.axis_index("s") * NC + lax.axis_index("c")
        base = wid * b_per_w

        pltpu.sync_copy(idx_hbm.at[pl.ds(base, b_per_w)], idx_v)

        def gather_dma(i, b):
            return pltpu.make_async_copy(
                table_hbm.at[idx_v.at[pl.ds(i * C, C)]], rows_v.at[b], gsems[b]
            )

        def store_dma(i, b):
            return pltpu.make_async_copy(
                rows_v.at[b], out_hbm.at[pl.ds(base + i * C, C)], osems[b]
            )

        for b in range(NBUF):
            gather_dma(b, b).start()

        def group(g, carry):
            for b in range(NBUF):
                i = g * NBUF + b
                gather_dma(i, b).wait()
                store_dma(i, b).start()
            for b in range(NBUF):
                i = g * NBUF + b
                store_dma(i, b).wait()
                gather_dma(i + NBUF, b).start()
            return carry

        lax.fori_loop(0, n_groups - 1, group, 0)

        for b in range(NBUF):
            i = (n_groups - 1) * NBUF + b
            gather_dma(i, b).wait()
            store_dma(i, b).start()
        for b in range(NBUF):
            i = (n_groups - 1) * NBUF + b
            store_dma(i, b).wait()

    return gather_kernel


def kernel(goal_encoding, embed_table):
    batch, hist = goal_encoding.shape
    v, d = embed_table.shape
    idx = goal_encoding.reshape(-1).astype(jnp.int32)
    out = _make_gather(batch * hist, v, d)(idx, embed_table)
    return out.reshape(batch, hist, d)
